# baseline (speedup 1.0000x reference)
import functools

import jax
import jax.numpy as jnp
from jax import lax
from jax.experimental import pallas as pl
from jax.experimental.pallas import tpu as pltpu
from jax.experimental.pallas import tpu_sc as plsc

N = 10000
E = 320000
D = 128

NC = 2
NS = 16
NW = NC * NS

C = 128
NBUF = 2
EPT = -(-E // NW // (C * NBUF)) * (C * NBUF)
E_PAD = EPT * NW
NCHUNK = EPT // C
E_ALLOC = E_PAD + NBUF * C

N_PAD = 10240
RPT = N_PAD // NS

_mesh = plsc.VectorSubcoreMesh(core_axis_name="c", subcore_axis_name="s",
                               num_cores=NC, num_subcores=NS)


@functools.partial(
    pl.kernel,
    out_type=jax.ShapeDtypeStruct((NC, N_PAD), jnp.float32),
    mesh=_mesh,
    scratch_types=[
        pltpu.VMEM((C,), jnp.int32),
        pltpu.VMEM((C,), jnp.float32),
        pltpu.VMEM((128,), jnp.float32),
        pltpu.VMEM_SHARED((N_PAD,), jnp.float32),
    ],
)
def _sc_degree(dst_hbm, out_hbm, didx, ones, zbuf, accum):
    c = lax.axis_index("c")
    s = lax.axis_index("s")
    wid = s * NC + c

    for j in range(8):
        zbuf[pl.ds(j * 16, 16)] = jnp.zeros((16,), jnp.float32)
        ones[pl.ds(j * 16, 16)] = jnp.ones((16,), jnp.float32)

    base = s * RPT

    def zloop(t, carry):
        pltpu.sync_copy(zbuf, accum.at[pl.ds(base + t * 128, 128)])
        return carry

    lax.fori_loop(0, RPT // 128, zloop, 0)
    plsc.subcore_barrier()

    eb = wid * EPT

    def eloop(t, carry):
        pltpu.sync_copy(dst_hbm.at[pl.ds(eb + t * C, C)], didx)
        pltpu.sync_copy(ones, accum.at[didx], add=True)
        return carry

    lax.fori_loop(0, NCHUNK, eloop, 0)
    plsc.subcore_barrier()

    pltpu.sync_copy(accum.at[pl.ds(base, RPT)], out_hbm.at[c, pl.ds(base, RPT)])


@functools.partial(
    pl.kernel,
    out_type=jax.ShapeDtypeStruct((NC, N_PAD, D), jnp.float32),
    mesh=_mesh,
    scratch_types=[
        [pltpu.VMEM((C,), jnp.int32) for _ in range(NBUF)],
        [pltpu.VMEM((C,), jnp.int32) for _ in range(NBUF)],
        [pltpu.VMEM((C, D), jnp.float32) for _ in range(NBUF)],
        pltpu.VMEM((16, D), jnp.float32),
        pltpu.VMEM_SHARED((N_PAD, D), jnp.float32),
        [pltpu.SemaphoreType.DMA for _ in range(NBUF)],
        pltpu.SemaphoreType.DMA,
    ],
)
def _sc_scatter(g_hbm, src_hbm, dst_hbm, out_hbm, sidx, didx, rows, zbuf,
                accum, sems, zsem):
    c = lax.axis_index("c")
    s = lax.axis_index("s")
    wid = s * NC + c

    for i in range(16):
        for j in range(8):
            zbuf[i, pl.ds(j * 16, 16)] = jnp.zeros((16,), jnp.float32)

    base = s * RPT

    def zstart(t, carry):
        pltpu.async_copy(zbuf, accum.at[pl.ds(base + t * 16, 16)], zsem)
        return carry

    lax.fori_loop(0, RPT // 16, zstart, 0)

    def zdrain(t, carry):
        pltpu.make_async_copy(zbuf, accum.at[pl.ds(base, 16)], zsem).wait()
        return carry

    lax.fori_loop(0, RPT // 16, zdrain, 0)
    plsc.subcore_barrier()

    eb = wid * EPT

    def load_idx(chunk, b):
        off = eb + chunk * C
        pltpu.sync_copy(src_hbm.at[pl.ds(off, C)], sidx[b])
        pltpu.sync_copy(dst_hbm.at[pl.ds(off, C)], didx[b])

    for b in range(NBUF):
        load_idx(b, b)
        pltpu.async_copy(g_hbm.at[sidx[b]], rows[b], sems[b])

    def eloop(t, carry):
        for b in range(NBUF):
            chunk = t * NBUF + b
            pltpu.make_async_copy(g_hbm.at[sidx[b]], rows[b], sems[b]).wait()
            pltpu.sync_copy(rows[b], accum.at[didx[b]], add=True)
            load_idx(chunk + NBUF, b)
            pltpu.async_copy(g_hbm.at[sidx[b]], rows[b], sems[b])
        return carry

    lax.fori_loop(0, NCHUNK // NBUF, eloop, 0, unroll=False)

    for b in range(NBUF):
        pltpu.make_async_copy(g_hbm.at[sidx[b]], rows[b], sems[b]).wait()

    plsc.subcore_barrier()
    pltpu.sync_copy(accum.at[pl.ds(base, RPT)],
                    out_hbm.at[c, pl.ds(base, RPT)])


BN = 2000


def _dinv(d0, d1):
    return lax.rsqrt(d0 + d1 + 1.0)


def _tc_pre_body(d0_ref, d1_ref, x_ref, w_ref, o_ref):
    dinv = _dinv(d0_ref[...], d1_ref[...])
    o_ref[...] = dinv * jnp.dot(x_ref[...], w_ref[...],
                                preferred_element_type=jnp.float32)


def _tc_mid_body(p0_ref, p1_ref, g_ref, d0_ref, d1_ref, b_ref, w_ref, o_ref):
    dinv = _dinv(d0_ref[...], d1_ref[...])
    h = dinv * (p0_ref[...] + p1_ref[...] + g_ref[...]) + b_ref[...]
    h = jnp.maximum(h, 0.0)
    o_ref[...] = dinv * jnp.dot(h, w_ref[...],
                                preferred_element_type=jnp.float32)


def _tc_fin_body(p0_ref, p1_ref, g_ref, d0_ref, d1_ref, b_ref, o_ref):
    dinv = _dinv(d0_ref[...], d1_ref[...])
    o_ref[...] = dinv * (p0_ref[...] + p1_ref[...] + g_ref[...]) + b_ref[...]


_row_spec = pl.BlockSpec((BN, D), lambda i: (i, 0))
_col_spec = pl.BlockSpec((BN, 1), lambda i: (i, 0))
_w_spec = pl.BlockSpec((D, D), lambda i: (0, 0))
_b_spec = pl.BlockSpec((1, D), lambda i: (0, 0))
_grid = (N // BN,)
_out_sds = jax.ShapeDtypeStruct((N, D), jnp.float32)

_tc_pre = pl.pallas_call(
    _tc_pre_body, grid=_grid,
    in_specs=[_col_spec, _col_spec, _row_spec, _w_spec],
    out_specs=_row_spec, out_shape=_out_sds)

_tc_mid = pl.pallas_call(
    _tc_mid_body, grid=_grid,
    in_specs=[_row_spec, _row_spec, _row_spec, _col_spec, _col_spec,
              _b_spec, _w_spec],
    out_specs=_row_spec, out_shape=_out_sds)

_tc_fin = pl.pallas_call(
    _tc_fin_body, grid=_grid,
    in_specs=[_row_spec, _row_spec, _row_spec, _col_spec, _col_spec, _b_spec],
    out_specs=_row_spec, out_shape=_out_sds)


def kernel(x, edge_index, W1, b1, W2, b2, W3, b3):
    src = edge_index[0].astype(jnp.int32)
    dst = edge_index[1].astype(jnp.int32)
    pad = E_ALLOC - E
    src_p = jnp.concatenate([src, jnp.zeros((pad,), jnp.int32)])
    dst_p = jnp.concatenate([dst, jnp.full((pad,), N_PAD - 1, jnp.int32)])

    degp = _sc_degree(dst_p)
    d0 = degp[0, :N, None]
    d1 = degp[1, :N, None]

    b1r = b1.reshape(1, D)
    b2r = b2.reshape(1, D)
    b3r = b3.reshape(1, D)

    g1 = _tc_pre(d0, d1, x, W1)
    p = _sc_scatter(g1, src_p, dst_p)
    g2 = _tc_mid(p[0, :N], p[1, :N], g1, d0, d1, b1r, W2)
    p = _sc_scatter(g2, src_p, dst_p)
    g3 = _tc_mid(p[0, :N], p[1, :N], g2, d0, d1, b2r, W3)
    p = _sc_scatter(g3, src_p, dst_p)
    return _tc_fin(p[0, :N], p[1, :N], g3, d0, d1, b3r)


# speedup vs baseline: 1.6935x; 1.6935x over previous
import functools

import jax
import jax.numpy as jnp
from jax import lax
from jax.experimental import pallas as pl
from jax.experimental.pallas import tpu as pltpu
from jax.experimental.pallas import tpu_sc as plsc

N = 10000
E = 320000
D = 128

NC = 2
NS = 16
NW = NC * NS

C = 128
NBUF = 2
CB = C * NBUF

F0_NUM, F0_DEN = 16, 25
EPP = E // NS
EPT0 = (EPP * F0_NUM // F0_DEN // CB) * CB
EPT1 = -(-(EPP - EPT0) // CB) * CB
E_PAD = NS * (EPT0 + EPT1)
E_ALLOC = E_PAD + NBUF * C

N_PAD = 10240
RPT = N_PAD // NS

EPT_DEG = -(-E // NW // C) * C

_mesh = plsc.VectorSubcoreMesh(core_axis_name="c", subcore_axis_name="s",
                               num_cores=NC, num_subcores=NS)


@functools.partial(
    pl.kernel,
    out_type=jax.ShapeDtypeStruct((NC, N_PAD), jnp.float32),
    mesh=_mesh,
    scratch_types=[
        pltpu.VMEM((C,), jnp.int32),
        pltpu.VMEM((C,), jnp.float32),
        pltpu.VMEM((128,), jnp.float32),
        pltpu.VMEM_SHARED((N_PAD,), jnp.float32),
    ],
)
def _sc_degree(dst_hbm, out_hbm, didx, ones, zbuf, accum):
    c = lax.axis_index("c")
    s = lax.axis_index("s")
    wid = s * NC + c

    for j in range(8):
        zbuf[pl.ds(j * 16, 16)] = jnp.zeros((16,), jnp.float32)
        ones[pl.ds(j * 16, 16)] = jnp.ones((16,), jnp.float32)

    base = s * RPT

    def zloop(t, carry):
        pltpu.sync_copy(zbuf, accum.at[pl.ds(base + t * 128, 128)])
        return carry

    lax.fori_loop(0, RPT // 128, zloop, 0)
    plsc.subcore_barrier()

    eb = wid * EPT_DEG

    def eloop(t, carry):
        pltpu.sync_copy(dst_hbm.at[pl.ds(eb + t * C, C)], didx)
        pltpu.sync_copy(ones, accum.at[didx], add=True)
        return carry

    lax.fori_loop(0, EPT_DEG // C, eloop, 0)
    plsc.subcore_barrier()

    pltpu.sync_copy(accum.at[pl.ds(base, RPT)], out_hbm.at[c, pl.ds(base, RPT)])


@functools.partial(
    pl.kernel,
    out_type=jax.ShapeDtypeStruct((NC, N_PAD, D), jnp.float32),
    mesh=_mesh,
    scratch_types=[
        [pltpu.VMEM((C,), jnp.int32) for _ in range(NBUF)],
        [pltpu.VMEM((C,), jnp.int32) for _ in range(NBUF)],
        [pltpu.VMEM((C, D), jnp.float32) for _ in range(NBUF)],
        pltpu.VMEM((16, D), jnp.float32),
        pltpu.VMEM_SHARED((N_PAD, D), jnp.float32),
        [pltpu.SemaphoreType.DMA for _ in range(NBUF)],
        pltpu.SemaphoreType.DMA,
    ],
)
def _sc_scatter(g_hbm, src_hbm, dst_hbm, out_hbm, sidx, didx, rows, zbuf,
                accum, sems, zsem):
    c = lax.axis_index("c")
    s = lax.axis_index("s")

    for i in range(16):
        for j in range(8):
            zbuf[i, pl.ds(j * 16, 16)] = jnp.zeros((16,), jnp.float32)

    base = s * RPT

    def zstart(t, carry):
        pltpu.async_copy(zbuf, accum.at[pl.ds(base + t * 16, 16)], zsem)
        return carry

    lax.fori_loop(0, RPT // 16, zstart, 0)

    def zdrain(t, carry):
        pltpu.make_async_copy(zbuf, accum.at[pl.ds(base, 16)], zsem).wait()
        return carry

    lax.fori_loop(0, RPT // 16, zdrain, 0)
    plsc.subcore_barrier()

    eb = jnp.where(c == 0, s * EPT0, NS * EPT0 + s * EPT1)
    niter = jnp.where(c == 0, EPT0 // CB, EPT1 // CB)

    def load_idx(off, b):
        pltpu.sync_copy(src_hbm.at[pl.ds(off, C)], sidx[b])
        pltpu.sync_copy(dst_hbm.at[pl.ds(off, C)], didx[b])

    for b in range(NBUF):
        load_idx(eb + b * C, b)
        pltpu.async_copy(g_hbm.at[sidx[b]], rows[b], sems[b])

    def eloop(t, carry):
        for b in range(NBUF):
            pltpu.make_async_copy(g_hbm.at[sidx[b]], rows[b], sems[b]).wait()
            pltpu.sync_copy(rows[b], accum.at[didx[b]], add=True)
            load_idx(eb + t * CB + (b + NBUF) * C, b)
            pltpu.async_copy(g_hbm.at[sidx[b]], rows[b], sems[b])
        return carry

    lax.fori_loop(0, niter, eloop, 0, unroll=False)

    for b in range(NBUF):
        pltpu.make_async_copy(g_hbm.at[sidx[b]], rows[b], sems[b]).wait()

    plsc.subcore_barrier()
    pltpu.sync_copy(accum.at[pl.ds(base, RPT)],
                    out_hbm.at[c, pl.ds(base, RPT)])


BN = 2000


def _dinv(d0, d1):
    return lax.rsqrt(d0 + d1 + 1.0)


def _tc_pre_body(d0_ref, d1_ref, x_ref, w_ref, o_ref):
    dinv = _dinv(d0_ref[...], d1_ref[...])
    o_ref[...] = dinv * jnp.dot(x_ref[...], w_ref[...],
                                preferred_element_type=jnp.float32)


def _tc_mid_body(p0_ref, p1_ref, g_ref, d0_ref, d1_ref, b_ref, w_ref, o_ref):
    dinv = _dinv(d0_ref[...], d1_ref[...])
    h = dinv * (p0_ref[...] + p1_ref[...] + g_ref[...]) + b_ref[...]
    h = jnp.maximum(h, 0.0)
    o_ref[...] = dinv * jnp.dot(h, w_ref[...],
                                preferred_element_type=jnp.float32)


def _tc_fin_body(p0_ref, p1_ref, g_ref, d0_ref, d1_ref, b_ref, o_ref):
    dinv = _dinv(d0_ref[...], d1_ref[...])
    o_ref[...] = dinv * (p0_ref[...] + p1_ref[...] + g_ref[...]) + b_ref[...]


_row_spec = pl.BlockSpec((BN, D), lambda i: (i, 0))
_col_spec = pl.BlockSpec((BN, 1), lambda i: (i, 0))
_w_spec = pl.BlockSpec((D, D), lambda i: (0, 0))
_b_spec = pl.BlockSpec((1, D), lambda i: (0, 0))
_grid = (N // BN,)
_out_sds = jax.ShapeDtypeStruct((N, D), jnp.float32)

_tc_pre = pl.pallas_call(
    _tc_pre_body, grid=_grid,
    in_specs=[_col_spec, _col_spec, _row_spec, _w_spec],
    out_specs=_row_spec, out_shape=_out_sds)

_tc_mid = pl.pallas_call(
    _tc_mid_body, grid=_grid,
    in_specs=[_row_spec, _row_spec, _row_spec, _col_spec, _col_spec,
              _b_spec, _w_spec],
    out_specs=_row_spec, out_shape=_out_sds)

_tc_fin = pl.pallas_call(
    _tc_fin_body, grid=_grid,
    in_specs=[_row_spec, _row_spec, _row_spec, _col_spec, _col_spec, _b_spec],
    out_specs=_row_spec, out_shape=_out_sds)


def kernel(x, edge_index, W1, b1, W2, b2, W3, b3):
    src = edge_index[0].astype(jnp.int32)
    dst = edge_index[1].astype(jnp.int32)
    pad = E_ALLOC - E
    src_p = jnp.concatenate([src, jnp.zeros((pad,), jnp.int32)])
    dst_p = jnp.concatenate([dst, jnp.full((pad,), N_PAD - 1, jnp.int32)])

    degp = _sc_degree(dst_p)
    d0 = degp[0, :N, None]
    d1 = degp[1, :N, None]

    b1r = b1.reshape(1, D)
    b2r = b2.reshape(1, D)
    b3r = b3.reshape(1, D)

    g1 = _tc_pre(d0, d1, x, W1)
    p = _sc_scatter(g1, src_p, dst_p)
    g2 = _tc_mid(p[0, :N], p[1, :N], g1, d0, d1, b1r, W2)
    p = _sc_scatter(g2, src_p, dst_p)
    g3 = _tc_mid(p[0, :N], p[1, :N], g2, d0, d1, b2r, W3)
    p = _sc_scatter(g3, src_p, dst_p)
    return _tc_fin(p[0, :N], p[1, :N], g3, d0, d1, b3r)


# speedup vs baseline: 1.7639x; 1.0416x over previous
import functools

import jax
import jax.numpy as jnp
from jax import lax
from jax.experimental import pallas as pl
from jax.experimental.pallas import tpu as pltpu
from jax.experimental.pallas import tpu_sc as plsc

N = 10000
E = 320000
D = 128

NC = 2
NS = 16
NW = NC * NS

C = 128
NBUF = 2
CB = C * NBUF

F0_NUM, F0_DEN = 88, 125
EPP = E // NS
EPT0 = (EPP * F0_NUM // F0_DEN // CB) * CB
EPT1 = -(-(EPP - EPT0) // CB) * CB
E_PAD = NS * (EPT0 + EPT1)
E_ALLOC = E_PAD + NBUF * C

N_PAD = 10240
RPT = N_PAD // NS

EPT_DEG = -(-E // NW // C) * C

_mesh = plsc.VectorSubcoreMesh(core_axis_name="c", subcore_axis_name="s",
                               num_cores=NC, num_subcores=NS)


@functools.partial(
    pl.kernel,
    out_type=jax.ShapeDtypeStruct((NC, N_PAD), jnp.float32),
    mesh=_mesh,
    scratch_types=[
        pltpu.VMEM((C,), jnp.int32),
        pltpu.VMEM((C,), jnp.float32),
        pltpu.VMEM((128,), jnp.float32),
        pltpu.VMEM_SHARED((N_PAD,), jnp.float32),
    ],
)
def _sc_degree(dst_hbm, out_hbm, didx, ones, zbuf, accum):
    c = lax.axis_index("c")
    s = lax.axis_index("s")
    wid = s * NC + c

    for j in range(8):
        zbuf[pl.ds(j * 16, 16)] = jnp.zeros((16,), jnp.float32)
        ones[pl.ds(j * 16, 16)] = jnp.ones((16,), jnp.float32)

    base = s * RPT

    def zloop(t, carry):
        pltpu.sync_copy(zbuf, accum.at[pl.ds(base + t * 128, 128)])
        return carry

    lax.fori_loop(0, RPT // 128, zloop, 0)
    plsc.subcore_barrier()

    eb = wid * EPT_DEG

    def eloop(t, carry):
        pltpu.sync_copy(dst_hbm.at[pl.ds(eb + t * C, C)], didx)
        pltpu.sync_copy(ones, accum.at[didx], add=True)
        return carry

    lax.fori_loop(0, EPT_DEG // C, eloop, 0)
    plsc.subcore_barrier()

    pltpu.sync_copy(accum.at[pl.ds(base, RPT)], out_hbm.at[c, pl.ds(base, RPT)])


@functools.partial(
    pl.kernel,
    out_type=jax.ShapeDtypeStruct((NC, N_PAD, D), jnp.float32),
    mesh=_mesh,
    scratch_types=[
        [pltpu.VMEM((C,), jnp.int32) for _ in range(NBUF)],
        [pltpu.VMEM((C,), jnp.int32) for _ in range(NBUF)],
        [pltpu.VMEM((C, D), jnp.float32) for _ in range(NBUF)],
        pltpu.VMEM((16, D), jnp.float32),
        pltpu.VMEM_SHARED((N_PAD, D), jnp.float32),
        [pltpu.SemaphoreType.DMA for _ in range(NBUF)],
        pltpu.SemaphoreType.DMA,
    ],
)
def _sc_scatter(g_hbm, src_hbm, dst_hbm, out_hbm, sidx, didx, rows, zbuf,
                accum, sems, zsem):
    c = lax.axis_index("c")
    s = lax.axis_index("s")

    for i in range(16):
        for j in range(8):
            zbuf[i, pl.ds(j * 16, 16)] = jnp.zeros((16,), jnp.float32)

    base = s * RPT

    def zstart(t, carry):
        pltpu.async_copy(zbuf, accum.at[pl.ds(base + t * 16, 16)], zsem)
        return carry

    lax.fori_loop(0, RPT // 16, zstart, 0)

    def zdrain(t, carry):
        pltpu.make_async_copy(zbuf, accum.at[pl.ds(base, 16)], zsem).wait()
        return carry

    lax.fori_loop(0, RPT // 16, zdrain, 0)
    plsc.subcore_barrier()

    eb = jnp.where(c == 0, s * EPT0, NS * EPT0 + s * EPT1)
    niter = jnp.where(c == 0, EPT0 // CB, EPT1 // CB)

    def load_idx(off, b):
        pltpu.sync_copy(src_hbm.at[pl.ds(off, C)], sidx[b])
        pltpu.sync_copy(dst_hbm.at[pl.ds(off, C)], didx[b])

    for b in range(NBUF):
        load_idx(eb + b * C, b)
        pltpu.async_copy(g_hbm.at[sidx[b]], rows[b], sems[b])

    def eloop(t, carry):
        for b in range(NBUF):
            pltpu.make_async_copy(g_hbm.at[sidx[b]], rows[b], sems[b]).wait()
            pltpu.sync_copy(rows[b], accum.at[didx[b]], add=True)
            load_idx(eb + t * CB + (b + NBUF) * C, b)
            pltpu.async_copy(g_hbm.at[sidx[b]], rows[b], sems[b])
        return carry

    lax.fori_loop(0, niter, eloop, 0, unroll=False)

    for b in range(NBUF):
        pltpu.make_async_copy(g_hbm.at[sidx[b]], rows[b], sems[b]).wait()

    plsc.subcore_barrier()
    pltpu.sync_copy(accum.at[pl.ds(base, RPT)],
                    out_hbm.at[c, pl.ds(base, RPT)])


BN = 2000


def _dinv(d0, d1):
    return lax.rsqrt(d0 + d1 + 1.0)


def _tc_pre_body(d0_ref, d1_ref, x_ref, w_ref, o_ref):
    dinv = _dinv(d0_ref[...], d1_ref[...])
    o_ref[...] = dinv * jnp.dot(x_ref[...], w_ref[...],
                                preferred_element_type=jnp.float32)


def _tc_mid_body(p0_ref, p1_ref, g_ref, d0_ref, d1_ref, b_ref, w_ref, o_ref):
    dinv = _dinv(d0_ref[...], d1_ref[...])
    h = dinv * (p0_ref[...] + p1_ref[...] + g_ref[...]) + b_ref[...]
    h = jnp.maximum(h, 0.0)
    o_ref[...] = dinv * jnp.dot(h, w_ref[...],
                                preferred_element_type=jnp.float32)


def _tc_fin_body(p0_ref, p1_ref, g_ref, d0_ref, d1_ref, b_ref, o_ref):
    dinv = _dinv(d0_ref[...], d1_ref[...])
    o_ref[...] = dinv * (p0_ref[...] + p1_ref[...] + g_ref[...]) + b_ref[...]


_row_spec = pl.BlockSpec((BN, D), lambda i: (i, 0))
_col_spec = pl.BlockSpec((BN, 1), lambda i: (i, 0))
_w_spec = pl.BlockSpec((D, D), lambda i: (0, 0))
_b_spec = pl.BlockSpec((1, D), lambda i: (0, 0))
_grid = (N // BN,)
_out_sds = jax.ShapeDtypeStruct((N, D), jnp.float32)

_tc_pre = pl.pallas_call(
    _tc_pre_body, grid=_grid,
    in_specs=[_col_spec, _col_spec, _row_spec, _w_spec],
    out_specs=_row_spec, out_shape=_out_sds)

_tc_mid = pl.pallas_call(
    _tc_mid_body, grid=_grid,
    in_specs=[_row_spec, _row_spec, _row_spec, _col_spec, _col_spec,
              _b_spec, _w_spec],
    out_specs=_row_spec, out_shape=_out_sds)

_tc_fin = pl.pallas_call(
    _tc_fin_body, grid=_grid,
    in_specs=[_row_spec, _row_spec, _row_spec, _col_spec, _col_spec, _b_spec],
    out_specs=_row_spec, out_shape=_out_sds)


def kernel(x, edge_index, W1, b1, W2, b2, W3, b3):
    src = edge_index[0].astype(jnp.int32)
    dst = edge_index[1].astype(jnp.int32)
    pad = E_ALLOC - E
    src_p = jnp.concatenate([src, jnp.zeros((pad,), jnp.int32)])
    dst_p = jnp.concatenate([dst, jnp.full((pad,), N_PAD - 1, jnp.int32)])

    degp = _sc_degree(dst_p)
    d0 = degp[0, :N, None]
    d1 = degp[1, :N, None]

    b1r = b1.reshape(1, D)
    b2r = b2.reshape(1, D)
    b3r = b3.reshape(1, D)

    g1 = _tc_pre(d0, d1, x, W1)
    p = _sc_scatter(g1, src_p, dst_p)
    g2 = _tc_mid(p[0, :N], p[1, :N], g1, d0, d1, b1r, W2)
    p = _sc_scatter(g2, src_p, dst_p)
    g3 = _tc_mid(p[0, :N], p[1, :N], g2, d0, d1, b2r, W3)
    p = _sc_scatter(g3, src_p, dst_p)
    return _tc_fin(p[0, :N], p[1, :N], g3, d0, d1, b3r)


# speedup vs baseline: 2.1517x; 1.2199x over previous
import functools

import jax
import jax.numpy as jnp
from jax import lax
from jax.experimental import pallas as pl
from jax.experimental.pallas import tpu as pltpu
from jax.experimental.pallas import tpu_sc as plsc

N = 10000
E = 320000
D = 128

NC = 2
NS = 16
NW = NC * NS

C = 96
NROT = 3
CB = C * NROT

A0, A1 = 49, 21
EPT0, EPT1 = A0 * CB, A1 * CB
E_PAD = NS * (EPT0 + EPT1)

N_PAD = 10240
RPT = N_PAD // NS

CDEG = 128
EPT_DEG = -(-E // NW // CDEG) * CDEG
E_ALLOC = max(E_PAD + CB, EPT_DEG * NW)

_mesh = plsc.VectorSubcoreMesh(core_axis_name="c", subcore_axis_name="s",
                               num_cores=NC, num_subcores=NS)


@functools.partial(
    pl.kernel,
    out_type=jax.ShapeDtypeStruct((NC, N_PAD), jnp.float32),
    mesh=_mesh,
    scratch_types=[
        pltpu.VMEM((CDEG,), jnp.int32),
        pltpu.VMEM((CDEG,), jnp.float32),
        pltpu.VMEM((128,), jnp.float32),
        pltpu.VMEM_SHARED((N_PAD,), jnp.float32),
    ],
)
def _sc_degree(dst_hbm, out_hbm, didx, ones, zbuf, accum):
    c = lax.axis_index("c")
    s = lax.axis_index("s")
    wid = s * NC + c

    for j in range(8):
        zbuf[pl.ds(j * 16, 16)] = jnp.zeros((16,), jnp.float32)
        ones[pl.ds(j * 16, 16)] = jnp.ones((16,), jnp.float32)

    base = s * RPT

    def zloop(t, carry):
        pltpu.sync_copy(zbuf, accum.at[pl.ds(base + t * 128, 128)])
        return carry

    lax.fori_loop(0, RPT // 128, zloop, 0)
    plsc.subcore_barrier()

    eb = wid * EPT_DEG

    def eloop(t, carry):
        pltpu.sync_copy(dst_hbm.at[pl.ds(eb + t * CDEG, CDEG)], didx)
        pltpu.sync_copy(ones, accum.at[didx], add=True)
        return carry

    lax.fori_loop(0, EPT_DEG // CDEG, eloop, 0)
    plsc.subcore_barrier()

    pltpu.sync_copy(accum.at[pl.ds(base, RPT)], out_hbm.at[c, pl.ds(base, RPT)])


@functools.partial(
    pl.kernel,
    out_type=jax.ShapeDtypeStruct((NC, N_PAD, D), jnp.float32),
    mesh=_mesh,
    scratch_types=[
        [pltpu.VMEM((C,), jnp.int32) for _ in range(NROT)],
        [pltpu.VMEM((C,), jnp.int32) for _ in range(NROT)],
        [pltpu.VMEM((C, D), jnp.float32) for _ in range(NROT)],
        pltpu.VMEM((16, D), jnp.float32),
        pltpu.VMEM_SHARED((N_PAD, D), jnp.float32),
        [pltpu.SemaphoreType.DMA for _ in range(NROT)],
        [pltpu.SemaphoreType.DMA for _ in range(NROT)],
        [pltpu.SemaphoreType.DMA for _ in range(NROT)],
        [pltpu.SemaphoreType.DMA for _ in range(NROT)],
        pltpu.SemaphoreType.DMA,
    ],
)
def _sc_scatter(g_hbm, src_hbm, dst_hbm, out_hbm, sidx, didx, rows, zbuf,
                accum, gsem, ssem, issem, idsem, zsem):
    c = lax.axis_index("c")
    s = lax.axis_index("s")

    for i in range(16):
        for j in range(8):
            zbuf[i, pl.ds(j * 16, 16)] = jnp.zeros((16,), jnp.float32)

    base = s * RPT

    def zstart(t, carry):
        pltpu.async_copy(zbuf, accum.at[pl.ds(base + t * 16, 16)], zsem)
        return carry

    lax.fori_loop(0, RPT // 16, zstart, 0)

    def zdrain(t, carry):
        pltpu.make_async_copy(zbuf, accum.at[pl.ds(base, 16)], zsem).wait()
        return carry

    lax.fori_loop(0, RPT // 16, zdrain, 0)
    plsc.subcore_barrier()

    eb = jnp.where(c == 0, s * EPT0, NS * EPT0 + s * EPT1)
    nb = jnp.where(c == 0, A0, A1)

    def sslice(ch):
        return src_hbm.at[pl.ds(eb + ch * C, C)]

    def dslice(ch):
        return dst_hbm.at[pl.ds(eb + ch * C, C)]

    pltpu.sync_copy(sslice(0), sidx[0])
    pltpu.sync_copy(sslice(1), sidx[1])
    pltpu.async_copy(dslice(0), didx[0], idsem[0])
    pltpu.async_copy(dslice(1), didx[1], idsem[1])
    pltpu.async_copy(dslice(2), didx[2], idsem[2])
    pltpu.async_copy(sslice(2), sidx[2], issem[2])
    pltpu.async_copy(g_hbm.at[sidx[0]], rows[0], gsem[0])
    pltpu.async_copy(g_hbm.at[sidx[1]], rows[1], gsem[1])

    def step(k, ch, first=False):
        b = k % NROT
        bn = (k + 2) % NROT
        pltpu.make_async_copy(dslice(ch), didx[b], idsem[b]).wait()
        pltpu.make_async_copy(g_hbm.at[sidx[b]], rows[b], gsem[b]).wait()
        pltpu.async_copy(rows[b], accum.at[didx[b]], ssem[b], add=True)
        pltpu.async_copy(sslice(ch + 3), sidx[b], issem[b])
        if not first:
            pltpu.make_async_copy(rows[bn], accum.at[didx[bn]],
                                  ssem[bn]).wait()
            pltpu.async_copy(dslice(ch + 2), didx[bn], idsem[bn])
        pltpu.make_async_copy(sslice(ch + 2), sidx[bn], issem[bn]).wait()
        pltpu.async_copy(g_hbm.at[sidx[bn]], rows[bn], gsem[bn])

    step(0, 0, first=True)
    step(1, 1)
    step(2, 2)

    def eloop(t, carry):
        for k in range(NROT):
            step(k, t * NROT + k)
        return carry

    lax.fori_loop(1, nb, eloop, 0, unroll=False)

    nch = nb * NROT
    pltpu.make_async_copy(g_hbm.at[sidx[0]], rows[0], gsem[0]).wait()
    pltpu.make_async_copy(g_hbm.at[sidx[1]], rows[1], gsem[1]).wait()
    pltpu.make_async_copy(rows[2], accum.at[didx[2]], ssem[2]).wait()
    pltpu.make_async_copy(sslice(nch + 2), sidx[2], issem[2]).wait()
    pltpu.make_async_copy(dslice(nch), didx[0], idsem[0]).wait()
    pltpu.make_async_copy(dslice(nch + 1), didx[1], idsem[1]).wait()

    plsc.subcore_barrier()
    pltpu.sync_copy(accum.at[pl.ds(base, RPT)],
                    out_hbm.at[c, pl.ds(base, RPT)])


BN = 2000


def _dinv(d0, d1):
    return lax.rsqrt(d0 + d1 + 1.0)


def _tc_pre_body(d0_ref, d1_ref, x_ref, w_ref, o_ref):
    dinv = _dinv(d0_ref[...], d1_ref[...])
    o_ref[...] = dinv * jnp.dot(x_ref[...], w_ref[...],
                                preferred_element_type=jnp.float32)


def _tc_mid_body(p0_ref, p1_ref, g_ref, d0_ref, d1_ref, b_ref, w_ref, o_ref):
    dinv = _dinv(d0_ref[...], d1_ref[...])
    h = dinv * (p0_ref[...] + p1_ref[...] + g_ref[...]) + b_ref[...]
    h = jnp.maximum(h, 0.0)
    o_ref[...] = dinv * jnp.dot(h, w_ref[...],
                                preferred_element_type=jnp.float32)


def _tc_fin_body(p0_ref, p1_ref, g_ref, d0_ref, d1_ref, b_ref, o_ref):
    dinv = _dinv(d0_ref[...], d1_ref[...])
    o_ref[...] = dinv * (p0_ref[...] + p1_ref[...] + g_ref[...]) + b_ref[...]


_row_spec = pl.BlockSpec((BN, D), lambda i: (i, 0))
_col_spec = pl.BlockSpec((BN, 1), lambda i: (i, 0))
_w_spec = pl.BlockSpec((D, D), lambda i: (0, 0))
_b_spec = pl.BlockSpec((1, D), lambda i: (0, 0))
_grid = (N // BN,)
_out_sds = jax.ShapeDtypeStruct((N, D), jnp.float32)

_tc_pre = pl.pallas_call(
    _tc_pre_body, grid=_grid,
    in_specs=[_col_spec, _col_spec, _row_spec, _w_spec],
    out_specs=_row_spec, out_shape=_out_sds)

_tc_mid = pl.pallas_call(
    _tc_mid_body, grid=_grid,
    in_specs=[_row_spec, _row_spec, _row_spec, _col_spec, _col_spec,
              _b_spec, _w_spec],
    out_specs=_row_spec, out_shape=_out_sds)

_tc_fin = pl.pallas_call(
    _tc_fin_body, grid=_grid,
    in_specs=[_row_spec, _row_spec, _row_spec, _col_spec, _col_spec, _b_spec],
    out_specs=_row_spec, out_shape=_out_sds)


def kernel(x, edge_index, W1, b1, W2, b2, W3, b3):
    src = edge_index[0].astype(jnp.int32)
    dst = edge_index[1].astype(jnp.int32)
    pad = E_ALLOC - E
    src_p = jnp.concatenate([src, jnp.zeros((pad,), jnp.int32)])
    dst_p = jnp.concatenate([dst, jnp.full((pad,), N_PAD - 1, jnp.int32)])

    degp = _sc_degree(dst_p)
    d0 = degp[0, :N, None]
    d1 = degp[1, :N, None]

    b1r = b1.reshape(1, D)
    b2r = b2.reshape(1, D)
    b3r = b3.reshape(1, D)

    g1 = _tc_pre(d0, d1, x, W1)
    p = _sc_scatter(g1, src_p, dst_p)
    g2 = _tc_mid(p[0, :N], p[1, :N], g1, d0, d1, b1r, W2)
    p = _sc_scatter(g2, src_p, dst_p)
    g3 = _tc_mid(p[0, :N], p[1, :N], g2, d0, d1, b2r, W3)
    p = _sc_scatter(g3, src_p, dst_p)
    return _tc_fin(p[0, :N], p[1, :N], g3, d0, d1, b3r)


# speedup vs baseline: 2.2155x; 1.0296x over previous
import functools

import jax
import jax.numpy as jnp
from jax import lax
from jax.experimental import pallas as pl
from jax.experimental.pallas import tpu as pltpu
from jax.experimental.pallas import tpu_sc as plsc

N = 10000
E = 320000
D = 128

NC = 2
NS = 16
NW = NC * NS

C = 96
NROT = 3
CB = C * NROT

A0, A1 = 54, 16
EPT0, EPT1 = A0 * CB, A1 * CB
E_PAD = NS * (EPT0 + EPT1)

N_PAD = 10240
RPT = N_PAD // NS

CDEG = 128
EPT_DEG = -(-E // NW // CDEG) * CDEG
E_ALLOC = max(E_PAD + CB, EPT_DEG * NW)

_mesh = plsc.VectorSubcoreMesh(core_axis_name="c", subcore_axis_name="s",
                               num_cores=NC, num_subcores=NS)


@functools.partial(
    pl.kernel,
    out_type=jax.ShapeDtypeStruct((NC, N_PAD), jnp.float32),
    mesh=_mesh,
    scratch_types=[
        pltpu.VMEM((CDEG,), jnp.int32),
        pltpu.VMEM((CDEG,), jnp.float32),
        pltpu.VMEM((128,), jnp.float32),
        pltpu.VMEM_SHARED((N_PAD,), jnp.float32),
    ],
)
def _sc_degree(dst_hbm, out_hbm, didx, ones, zbuf, accum):
    c = lax.axis_index("c")
    s = lax.axis_index("s")
    wid = s * NC + c

    for j in range(8):
        zbuf[pl.ds(j * 16, 16)] = jnp.zeros((16,), jnp.float32)
        ones[pl.ds(j * 16, 16)] = jnp.ones((16,), jnp.float32)

    base = s * RPT

    def zloop(t, carry):
        pltpu.sync_copy(zbuf, accum.at[pl.ds(base + t * 128, 128)])
        return carry

    lax.fori_loop(0, RPT // 128, zloop, 0)
    plsc.subcore_barrier()

    eb = wid * EPT_DEG

    def eloop(t, carry):
        pltpu.sync_copy(dst_hbm.at[pl.ds(eb + t * CDEG, CDEG)], didx)
        pltpu.sync_copy(ones, accum.at[didx], add=True)
        return carry

    lax.fori_loop(0, EPT_DEG // CDEG, eloop, 0)
    plsc.subcore_barrier()

    pltpu.sync_copy(accum.at[pl.ds(base, RPT)], out_hbm.at[c, pl.ds(base, RPT)])


@functools.partial(
    pl.kernel,
    out_type=jax.ShapeDtypeStruct((NC, N_PAD, D), jnp.float32),
    mesh=_mesh,
    scratch_types=[
        [pltpu.VMEM((C,), jnp.int32) for _ in range(NROT)],
        [pltpu.VMEM((C,), jnp.int32) for _ in range(NROT)],
        [pltpu.VMEM((C, D), jnp.float32) for _ in range(NROT)],
        pltpu.VMEM((16, D), jnp.float32),
        pltpu.VMEM_SHARED((N_PAD, D), jnp.float32),
        [pltpu.SemaphoreType.DMA for _ in range(NROT)],
        [pltpu.SemaphoreType.DMA for _ in range(NROT)],
        [pltpu.SemaphoreType.DMA for _ in range(NROT)],
        [pltpu.SemaphoreType.DMA for _ in range(NROT)],
        pltpu.SemaphoreType.DMA,
    ],
)
def _sc_scatter(g_hbm, src_hbm, dst_hbm, out_hbm, sidx, didx, rows, zbuf,
                accum, gsem, ssem, issem, idsem, zsem):
    c = lax.axis_index("c")
    s = lax.axis_index("s")

    for i in range(16):
        for j in range(8):
            zbuf[i, pl.ds(j * 16, 16)] = jnp.zeros((16,), jnp.float32)

    base = s * RPT

    def zstart(t, carry):
        pltpu.async_copy(zbuf, accum.at[pl.ds(base + t * 16, 16)], zsem)
        return carry

    lax.fori_loop(0, RPT // 16, zstart, 0)

    def zdrain(t, carry):
        pltpu.make_async_copy(zbuf, accum.at[pl.ds(base, 16)], zsem).wait()
        return carry

    lax.fori_loop(0, RPT // 16, zdrain, 0)
    plsc.subcore_barrier()

    eb = jnp.where(c == 0, s * EPT0, NS * EPT0 + s * EPT1)
    nb = jnp.where(c == 0, A0, A1)

    def sslice(ch):
        return src_hbm.at[pl.ds(eb + ch * C, C)]

    def dslice(ch):
        return dst_hbm.at[pl.ds(eb + ch * C, C)]

    pltpu.sync_copy(sslice(0), sidx[0])
    pltpu.sync_copy(sslice(1), sidx[1])
    pltpu.async_copy(dslice(0), didx[0], idsem[0])
    pltpu.async_copy(dslice(1), didx[1], idsem[1])
    pltpu.async_copy(dslice(2), didx[2], idsem[2])
    pltpu.async_copy(sslice(2), sidx[2], issem[2])
    pltpu.async_copy(g_hbm.at[sidx[0]], rows[0], gsem[0])
    pltpu.async_copy(g_hbm.at[sidx[1]], rows[1], gsem[1])

    def step(k, ch, first=False):
        b = k % NROT
        bn = (k + 2) % NROT
        pltpu.make_async_copy(dslice(ch), didx[b], idsem[b]).wait()
        pltpu.make_async_copy(g_hbm.at[sidx[b]], rows[b], gsem[b]).wait()
        pltpu.async_copy(rows[b], accum.at[didx[b]], ssem[b], add=True)
        pltpu.async_copy(sslice(ch + 3), sidx[b], issem[b])
        if not first:
            pltpu.make_async_copy(rows[bn], accum.at[didx[bn]],
                                  ssem[bn]).wait()
            pltpu.async_copy(dslice(ch + 2), didx[bn], idsem[bn])
        pltpu.make_async_copy(sslice(ch + 2), sidx[bn], issem[bn]).wait()
        pltpu.async_copy(g_hbm.at[sidx[bn]], rows[bn], gsem[bn])

    step(0, 0, first=True)
    step(1, 1)
    step(2, 2)

    def eloop(t, carry):
        for k in range(NROT):
            step(k, t * NROT + k)
        return carry

    lax.fori_loop(1, nb, eloop, 0, unroll=False)

    nch = nb * NROT
    pltpu.make_async_copy(g_hbm.at[sidx[0]], rows[0], gsem[0]).wait()
    pltpu.make_async_copy(g_hbm.at[sidx[1]], rows[1], gsem[1]).wait()
    pltpu.make_async_copy(rows[2], accum.at[didx[2]], ssem[2]).wait()
    pltpu.make_async_copy(sslice(nch + 2), sidx[2], issem[2]).wait()
    pltpu.make_async_copy(dslice(nch), didx[0], idsem[0]).wait()
    pltpu.make_async_copy(dslice(nch + 1), didx[1], idsem[1]).wait()

    plsc.subcore_barrier()
    pltpu.sync_copy(accum.at[pl.ds(base, RPT)],
                    out_hbm.at[c, pl.ds(base, RPT)])


BN = 2000


def _dinv(d0, d1):
    return lax.rsqrt(d0 + d1 + 1.0)


def _tc_pre_body(d0_ref, d1_ref, x_ref, w_ref, o_ref):
    dinv = _dinv(d0_ref[...], d1_ref[...])
    o_ref[...] = dinv * jnp.dot(x_ref[...], w_ref[...],
                                preferred_element_type=jnp.float32)


def _tc_mid_body(p0_ref, p1_ref, g_ref, d0_ref, d1_ref, b_ref, w_ref, o_ref):
    dinv = _dinv(d0_ref[...], d1_ref[...])
    h = dinv * (p0_ref[...] + p1_ref[...] + g_ref[...]) + b_ref[...]
    h = jnp.maximum(h, 0.0)
    o_ref[...] = dinv * jnp.dot(h, w_ref[...],
                                preferred_element_type=jnp.float32)


def _tc_fin_body(p0_ref, p1_ref, g_ref, d0_ref, d1_ref, b_ref, o_ref):
    dinv = _dinv(d0_ref[...], d1_ref[...])
    o_ref[...] = dinv * (p0_ref[...] + p1_ref[...] + g_ref[...]) + b_ref[...]


_row_spec = pl.BlockSpec((BN, D), lambda i: (i, 0))
_col_spec = pl.BlockSpec((BN, 1), lambda i: (i, 0))
_w_spec = pl.BlockSpec((D, D), lambda i: (0, 0))
_b_spec = pl.BlockSpec((1, D), lambda i: (0, 0))
_grid = (N // BN,)
_out_sds = jax.ShapeDtypeStruct((N, D), jnp.float32)

_tc_pre = pl.pallas_call(
    _tc_pre_body, grid=_grid,
    in_specs=[_col_spec, _col_spec, _row_spec, _w_spec],
    out_specs=_row_spec, out_shape=_out_sds)

_tc_mid = pl.pallas_call(
    _tc_mid_body, grid=_grid,
    in_specs=[_row_spec, _row_spec, _row_spec, _col_spec, _col_spec,
              _b_spec, _w_spec],
    out_specs=_row_spec, out_shape=_out_sds)

_tc_fin = pl.pallas_call(
    _tc_fin_body, grid=_grid,
    in_specs=[_row_spec, _row_spec, _row_spec, _col_spec, _col_spec, _b_spec],
    out_specs=_row_spec, out_shape=_out_sds)


def kernel(x, edge_index, W1, b1, W2, b2, W3, b3):
    src = edge_index[0].astype(jnp.int32)
    dst = edge_index[1].astype(jnp.int32)
    pad = E_ALLOC - E
    src_p = jnp.concatenate([src, jnp.zeros((pad,), jnp.int32)])
    dst_p = jnp.concatenate([dst, jnp.full((pad,), N_PAD - 1, jnp.int32)])

    degp = _sc_degree(dst_p)
    d0 = degp[0, :N, None]
    d1 = degp[1, :N, None]

    b1r = b1.reshape(1, D)
    b2r = b2.reshape(1, D)
    b3r = b3.reshape(1, D)

    g1 = _tc_pre(d0, d1, x, W1)
    p = _sc_scatter(g1, src_p, dst_p)
    g2 = _tc_mid(p[0, :N], p[1, :N], g1, d0, d1, b1r, W2)
    p = _sc_scatter(g2, src_p, dst_p)
    g3 = _tc_mid(p[0, :N], p[1, :N], g2, d0, d1, b2r, W3)
    p = _sc_scatter(g3, src_p, dst_p)
    return _tc_fin(p[0, :N], p[1, :N], g3, d0, d1, b3r)


# speedup vs baseline: 2.4076x; 1.0867x over previous
import functools

import jax
import jax.numpy as jnp
from jax import lax
from jax.experimental import pallas as pl
from jax.experimental.pallas import tpu as pltpu
from jax.experimental.pallas import tpu_sc as plsc

N = 10000
E = 320000
D = 128

NC = 2
NS = 16
NW = NC * NS

C = 96
NROT = 3
CB = C * NROT

A0, A1 = 56, 14
EPT0, EPT1 = A0 * CB, A1 * CB
E_PAD = NS * (EPT0 + EPT1)

N_PAD = 10240
RPT = N_PAD // NS

CDEG = 128
EPT_DEG = -(-E // NW // (3 * CDEG)) * (3 * CDEG)
NCH_DEG = EPT_DEG // CDEG
E_ALLOC = max(E_PAD + CB, EPT_DEG * NW + 2 * CDEG)

_mesh = plsc.VectorSubcoreMesh(core_axis_name="c", subcore_axis_name="s",
                               num_cores=NC, num_subcores=NS)


@functools.partial(
    pl.kernel,
    out_type=jax.ShapeDtypeStruct((NC, N_PAD), jnp.float32),
    mesh=_mesh,
    scratch_types=[
        [pltpu.VMEM((CDEG,), jnp.int32) for _ in range(3)],
        pltpu.VMEM((CDEG,), jnp.float32),
        pltpu.VMEM((128,), jnp.float32),
        pltpu.VMEM_SHARED((N_PAD,), jnp.float32),
        [pltpu.SemaphoreType.DMA for _ in range(3)],
        [pltpu.SemaphoreType.DMA for _ in range(3)],
    ],
)
def _sc_degree(dst_hbm, out_hbm, didx, ones, zbuf, accum, dsem, ssem):
    c = lax.axis_index("c")
    s = lax.axis_index("s")
    wid = s * NC + c

    for j in range(8):
        zbuf[pl.ds(j * 16, 16)] = jnp.zeros((16,), jnp.float32)
        ones[pl.ds(j * 16, 16)] = jnp.ones((16,), jnp.float32)

    base = s * RPT

    def zloop(t, carry):
        pltpu.sync_copy(zbuf, accum.at[pl.ds(base + t * 128, 128)])
        return carry

    lax.fori_loop(0, RPT // 128, zloop, 0)
    plsc.subcore_barrier()

    eb = wid * EPT_DEG

    def dslice(ch):
        return dst_hbm.at[pl.ds(eb + ch * CDEG, CDEG)]

    for b in range(3):
        pltpu.async_copy(dslice(b), didx[b], dsem[b])

    def step(k, ch, first=False):
        b = k % 3
        bn = (k + 2) % 3
        pltpu.make_async_copy(dslice(ch), didx[b], dsem[b]).wait()
        pltpu.async_copy(ones, accum.at[didx[b]], ssem[b], add=True)
        if not first:
            pltpu.make_async_copy(ones, accum.at[didx[bn]], ssem[bn]).wait()
            pltpu.async_copy(dslice(ch + 2), didx[bn], dsem[bn])

    step(0, 0, first=True)
    step(1, 1)
    step(2, 2)

    def eloop(t, carry):
        for k in range(3):
            step(k, t * 3 + k)
        return carry

    lax.fori_loop(1, NCH_DEG // 3, eloop, 0, unroll=False)

    pltpu.make_async_copy(ones, accum.at[didx[(NCH_DEG - 1) % 3]],
                          ssem[(NCH_DEG - 1) % 3]).wait()
    pltpu.make_async_copy(dslice(NCH_DEG), didx[NCH_DEG % 3],
                          dsem[NCH_DEG % 3]).wait()
    pltpu.make_async_copy(dslice(NCH_DEG + 1), didx[(NCH_DEG + 1) % 3],
                          dsem[(NCH_DEG + 1) % 3]).wait()
    plsc.subcore_barrier()

    pltpu.sync_copy(accum.at[pl.ds(base, RPT)], out_hbm.at[c, pl.ds(base, RPT)])


@functools.partial(
    pl.kernel,
    out_type=jax.ShapeDtypeStruct((NC, N_PAD, D), jnp.float32),
    mesh=_mesh,
    scratch_types=[
        [pltpu.VMEM((C,), jnp.int32) for _ in range(NROT)],
        [pltpu.VMEM((C,), jnp.int32) for _ in range(NROT)],
        [pltpu.VMEM((C, D), jnp.float32) for _ in range(NROT)],
        pltpu.VMEM((16, D), jnp.float32),
        pltpu.VMEM_SHARED((N_PAD, D), jnp.float32),
        [pltpu.SemaphoreType.DMA for _ in range(NROT)],
        [pltpu.SemaphoreType.DMA for _ in range(NROT)],
        [pltpu.SemaphoreType.DMA for _ in range(NROT)],
        [pltpu.SemaphoreType.DMA for _ in range(NROT)],
        pltpu.SemaphoreType.DMA,
    ],
)
def _sc_scatter(g_hbm, src_hbm, dst_hbm, out_hbm, sidx, didx, rows, zbuf,
                accum, gsem, ssem, issem, idsem, zsem):
    c = lax.axis_index("c")
    s = lax.axis_index("s")

    for i in range(16):
        for j in range(8):
            zbuf[i, pl.ds(j * 16, 16)] = jnp.zeros((16,), jnp.float32)

    base = s * RPT

    def zstart(t, carry):
        pltpu.async_copy(zbuf, accum.at[pl.ds(base + t * 16, 16)], zsem)
        return carry

    lax.fori_loop(0, RPT // 16, zstart, 0)

    def zdrain(t, carry):
        pltpu.make_async_copy(zbuf, accum.at[pl.ds(base, 16)], zsem).wait()
        return carry

    lax.fori_loop(0, RPT // 16, zdrain, 0)
    plsc.subcore_barrier()

    eb = jnp.where(c == 0, s * EPT0, NS * EPT0 + s * EPT1)
    nb = jnp.where(c == 0, A0, A1)

    def sslice(ch):
        return src_hbm.at[pl.ds(eb + ch * C, C)]

    def dslice(ch):
        return dst_hbm.at[pl.ds(eb + ch * C, C)]

    pltpu.sync_copy(sslice(0), sidx[0])
    pltpu.sync_copy(sslice(1), sidx[1])
    pltpu.async_copy(dslice(0), didx[0], idsem[0])
    pltpu.async_copy(dslice(1), didx[1], idsem[1])
    pltpu.async_copy(dslice(2), didx[2], idsem[2])
    pltpu.async_copy(sslice(2), sidx[2], issem[2])
    pltpu.async_copy(g_hbm.at[sidx[0]], rows[0], gsem[0])
    pltpu.async_copy(g_hbm.at[sidx[1]], rows[1], gsem[1])

    def step(k, ch, first=False):
        b = k % NROT
        bn = (k + 2) % NROT
        pltpu.make_async_copy(dslice(ch), didx[b], idsem[b]).wait()
        pltpu.make_async_copy(g_hbm.at[sidx[b]], rows[b], gsem[b]).wait()
        pltpu.async_copy(rows[b], accum.at[didx[b]], ssem[b], add=True)
        pltpu.async_copy(sslice(ch + 3), sidx[b], issem[b])
        if not first:
            pltpu.make_async_copy(rows[bn], accum.at[didx[bn]],
                                  ssem[bn]).wait()
            pltpu.async_copy(dslice(ch + 2), didx[bn], idsem[bn])
        pltpu.make_async_copy(sslice(ch + 2), sidx[bn], issem[bn]).wait()
        pltpu.async_copy(g_hbm.at[sidx[bn]], rows[bn], gsem[bn])

    step(0, 0, first=True)
    step(1, 1)
    step(2, 2)

    def eloop(t, carry):
        for k in range(NROT):
            step(k, t * NROT + k)
        return carry

    lax.fori_loop(1, nb, eloop, 0, unroll=False)

    nch = nb * NROT
    pltpu.make_async_copy(g_hbm.at[sidx[0]], rows[0], gsem[0]).wait()
    pltpu.make_async_copy(g_hbm.at[sidx[1]], rows[1], gsem[1]).wait()
    pltpu.make_async_copy(rows[2], accum.at[didx[2]], ssem[2]).wait()
    pltpu.make_async_copy(sslice(nch + 2), sidx[2], issem[2]).wait()
    pltpu.make_async_copy(dslice(nch), didx[0], idsem[0]).wait()
    pltpu.make_async_copy(dslice(nch + 1), didx[1], idsem[1]).wait()

    plsc.subcore_barrier()
    pltpu.sync_copy(accum.at[pl.ds(base, RPT)],
                    out_hbm.at[c, pl.ds(base, RPT)])


BN = 2000


def _dinv(dp):
    return lax.rsqrt(dp[0] + dp[1] + 1.0)


def _tc_pre_body(dp_ref, x_ref, w_ref, o_ref):
    dinv = _dinv(dp_ref[...])
    o_ref[...] = dinv * jnp.dot(x_ref[...], w_ref[...],
                                preferred_element_type=jnp.float32)


def _tc_mid_body(p_ref, g_ref, dp_ref, b_ref, w_ref, o_ref):
    dinv = _dinv(dp_ref[...])
    pv = p_ref[...]
    h = dinv * (pv[0] + pv[1] + g_ref[...]) + b_ref[...]
    h = jnp.maximum(h, 0.0)
    o_ref[...] = dinv * jnp.dot(h, w_ref[...],
                                preferred_element_type=jnp.float32)


def _tc_fin_body(p_ref, g_ref, dp_ref, b_ref, o_ref):
    dinv = _dinv(dp_ref[...])
    pv = p_ref[...]
    o_ref[...] = dinv * (pv[0] + pv[1] + g_ref[...]) + b_ref[...]


_row_spec = pl.BlockSpec((BN, D), lambda i: (i, 0))
_p_spec = pl.BlockSpec((NC, BN, D), lambda i: (0, i, 0))
_deg_spec = pl.BlockSpec((NC, BN, 1), lambda i: (0, i, 0))
_w_spec = pl.BlockSpec((D, D), lambda i: (0, 0))
_b_spec = pl.BlockSpec((1, D), lambda i: (0, 0))
_grid = (N // BN,)
_out_sds = jax.ShapeDtypeStruct((N, D), jnp.float32)

_tc_pre = pl.pallas_call(
    _tc_pre_body, grid=_grid,
    in_specs=[_deg_spec, _row_spec, _w_spec],
    out_specs=_row_spec, out_shape=_out_sds)

_tc_mid = pl.pallas_call(
    _tc_mid_body, grid=_grid,
    in_specs=[_p_spec, _row_spec, _deg_spec, _b_spec, _w_spec],
    out_specs=_row_spec, out_shape=_out_sds)

_tc_fin = pl.pallas_call(
    _tc_fin_body, grid=_grid,
    in_specs=[_p_spec, _row_spec, _deg_spec, _b_spec],
    out_specs=_row_spec, out_shape=_out_sds)


def kernel(x, edge_index, W1, b1, W2, b2, W3, b3):
    src = edge_index[0].astype(jnp.int32)
    dst = edge_index[1].astype(jnp.int32)
    pad = E_ALLOC - E
    src_p = jnp.concatenate([src, jnp.zeros((pad,), jnp.int32)])
    dst_p = jnp.concatenate([dst, jnp.full((pad,), N_PAD - 1, jnp.int32)])

    degp = _sc_degree(dst_p)[:, :, None]

    b1r = b1.reshape(1, D)
    b2r = b2.reshape(1, D)
    b3r = b3.reshape(1, D)

    g1 = _tc_pre(degp, x, W1)
    p = _sc_scatter(g1, src_p, dst_p)
    g2 = _tc_mid(p, g1, degp, b1r, W2)
    p = _sc_scatter(g2, src_p, dst_p)
    g3 = _tc_mid(p, g2, degp, b2r, W3)
    p = _sc_scatter(g3, src_p, dst_p)
    return _tc_fin(p, g3, degp, b3r)
